# Initial kernel scaffold; baseline (speedup 1.0000x reference)
import jax
import jax.numpy as jnp
from jax.experimental import pallas as pl


def kernel(init_vertices, init_springs, init_rest_lengths, init_masses, spring_Y_log, collide_elas, collide_fric):
    raise NotImplementedError("write your pallas kernel here")


# speedup vs baseline: 3.1039x; 3.1039x over previous
import math

import jax
import jax.numpy as jnp
from jax import lax
from jax.experimental import pallas as pl
from jax.experimental.pallas import tpu as pltpu
from jax.experimental.pallas import tpu_sc as plsc

N_VERT = 50000
N_SPR = 800000
DT = 0.001
NUM_SUBSTEPS = 10
DASHPOT_DAMPING = 0.1
DRAG_DAMPING = 0.5
COLLISION_DIST = 0.05

NW = 32
VP = 50176
RPT = VP // NW
EB = 2048
NBLK = 27
NPT = EB * NBLK
KSUB = EB // 128
XVW = 16
SF_PAD = 8192
NSF = N_SPR + SF_PAD

_DAMP = math.exp(-DT * DRAG_DAMPING)
_LANES = 16


def _rsqrt(x):
    xs = jnp.maximum(x, jnp.float32(1e-30))
    i = lax.bitcast_convert_type(xs, jnp.int32)
    i = jnp.int32(0x5F3759DF) - lax.shift_right_logical(i, 1)
    y = lax.bitcast_convert_type(i, jnp.float32)
    for _ in range(3):
        y = y * (jnp.float32(1.5) - jnp.float32(0.5) * xs * y * y)
    return y


def _c(v):
    return jnp.full((_LANES,), v, jnp.int32)


def _make_step(emit_sf: bool):
    mesh = plsc.VectorSubcoreMesh(core_axis_name="c", subcore_axis_name="s",
                                  num_cores=2, num_subcores=16)
    out_type = [jax.ShapeDtypeStruct((VP, XVW), jnp.float32)]
    if emit_sf:
        out_type.append(jax.ShapeDtypeStruct((NSF, 8), jnp.float32))
    scratch = [
        pltpu.VMEM((KSUB, 128), jnp.int32),
        pltpu.VMEM((KSUB, 128), jnp.int32),
        pltpu.VMEM((EB, 4), jnp.float32),
        pltpu.VMEM((EB, XVW), jnp.float32),
        pltpu.VMEM((EB, 8), jnp.float32),
        pltpu.VMEM((RPT * 4,), jnp.float32),
        pltpu.VMEM((RPT, XVW), jnp.float32),
        pltpu.VMEM((32,), jnp.float32),
        pltpu.SemaphoreType.DMA,
    ]

    def body(xv_in, srcg, ebg, osg, z4, prm, *rest):
        if emit_sf:
            xv_out, sf_out = rest[0], rest[1]
            scr = rest[2:]
        else:
            xv_out = rest[0]
            scr = rest[1:]
        (i1buf, osbuf, ebuf, rows1, sfbuf, accf, xvbuf, pbuf, sem) = scr
        cid = lax.axis_index("c")
        sid = lax.axis_index("s")
        wid = cid * 16 + sid
        iota = lax.iota(jnp.int32, _LANES)

        pltpu.sync_copy(xv_in.at[pl.ds(wid * RPT, RPT)], xvbuf)
        pltpu.sync_copy(z4.at[pl.ds(wid * RPT * 4, RPT * 4)], accf)
        pltpu.sync_copy(prm, pbuf)

        def block(b, carry):
            pltpu.sync_copy(srcg.at[wid, pl.ds(b * KSUB, KSUB)], i1buf)
            pltpu.sync_copy(ebg.at[wid, pl.ds(b * EB, EB)], ebuf)
            if emit_sf:
                pltpu.sync_copy(osg.at[wid, pl.ds(b * KSUB, KSUB)], osbuf)
            cps = [pltpu.async_copy(
                       xv_in.at[i1buf.at[kb]],
                       rows1.at[pl.ds(kb * 128, 128)], sem)
                   for kb in range(KSUB)]
            for cp in cps:
                cp.wait()

            def vec(j, carry2):
                r = j * _LANES + iota
                dstl = plsc.load_gather(ebuf, [r, _c(0)]).astype(jnp.int32)
                ylv = plsc.load_gather(ebuf, [r, _c(1)])
                rv = plsc.load_gather(ebuf, [r, _c(2)])
                x1x = plsc.load_gather(xvbuf, [dstl, _c(0)])
                x1y = plsc.load_gather(xvbuf, [dstl, _c(1)])
                x1z = plsc.load_gather(xvbuf, [dstl, _c(2)])
                v1x = plsc.load_gather(xvbuf, [dstl, _c(3)])
                v1y = plsc.load_gather(xvbuf, [dstl, _c(4)])
                v1z = plsc.load_gather(xvbuf, [dstl, _c(5)])
                x2x = plsc.load_gather(rows1, [r, _c(0)])
                x2y = plsc.load_gather(rows1, [r, _c(1)])
                x2z = plsc.load_gather(rows1, [r, _c(2)])
                v2x = plsc.load_gather(rows1, [r, _c(3)])
                v2y = plsc.load_gather(rows1, [r, _c(4)])
                v2z = plsc.load_gather(rows1, [r, _c(5)])
                Y = jnp.exp(ylv)
                ir = jnp.float32(1.0) / rv
                dx = x2x - x1x
                dy = x2y - x1y
                dz = x2z - x1z
                L2 = dx * dx + dy * dy + dz * dz
                ri = _rsqrt(L2)
                L = L2 * ri
                ddx = dx * ri
                ddy = dy * ri
                ddz = dz * ri
                scoef = Y * (L * ir - jnp.float32(1.0))
                vrel = ((v2x - v1x) * ddx + (v2y - v1y) * ddy
                        + (v2z - v1z) * ddz)
                fc = scoef + jnp.float32(DASHPOT_DAMPING) * vrel
                a4 = dstl * 4
                plsc.addupdate_scatter(accf, [a4], fc * ddx)
                plsc.addupdate_scatter(accf, [a4 + 1], fc * ddy)
                plsc.addupdate_scatter(accf, [a4 + 2], fc * ddz)
                if emit_sf:
                    plsc.store_scatter(sfbuf, [r, _c(0)], scoef * ddx)
                    plsc.store_scatter(sfbuf, [r, _c(1)], scoef * ddy)
                    plsc.store_scatter(sfbuf, [r, _c(2)], scoef * ddz)
                return carry2

            lax.fori_loop(0, EB // _LANES, vec, 0)

            if emit_sf:
                sps = [pltpu.async_copy(
                           sfbuf.at[pl.ds(kb * 128, 128)],
                           sf_out.at[osbuf.at[kb]], sem)
                       for kb in range(KSUB)]
                for cp in sps:
                    cp.wait()
            return carry

        lax.fori_loop(0, NBLK, block, 0)

        elas = jnp.clip(pbuf[pl.ds(0, _LANES)], 0.0, 1.0)
        fric = jnp.clip(pbuf[pl.ds(16, _LANES)], 0.0, 1.0)

        def vrow(j, carry2):
            r = j * _LANES + iota
            r4 = r * 4
            fx = plsc.load_gather(accf, [r4])
            fy = plsc.load_gather(accf, [r4 + 1])
            fz = plsc.load_gather(accf, [r4 + 2])
            xx = plsc.load_gather(xvbuf, [r, _c(0)])
            xy = plsc.load_gather(xvbuf, [r, _c(1)])
            xz = plsc.load_gather(xvbuf, [r, _c(2)])
            vx = plsc.load_gather(xvbuf, [r, _c(3)])
            vy = plsc.load_gather(xvbuf, [r, _c(4)])
            vz = plsc.load_gather(xvbuf, [r, _c(5)])
            m = plsc.load_gather(xvbuf, [r, _c(6)])
            minv = jnp.float32(1.0) / m
            nvx = (vx + jnp.float32(DT) * fx * minv) * jnp.float32(_DAMP)
            nvy = (vy + jnp.float32(DT) * fy * minv) * jnp.float32(_DAMP)
            nvz = ((vz + jnp.float32(DT) * (fz * minv - jnp.float32(9.8)))
                   * jnp.float32(_DAMP))
            nxx = xx + jnp.float32(DT) * nvx
            nxy = xy + jnp.float32(DT) * nvy
            nxz = xz + jnp.float32(DT) * nvz
            hit = (nxz < jnp.float32(COLLISION_DIST)) & (nvz < jnp.float32(0.0))
            vt2 = nvx * nvx + nvy * nvy
            vtn = vt2 * _rsqrt(vt2) + jnp.float32(1e-8)
            a = jnp.maximum(
                jnp.float32(0.0),
                jnp.float32(1.0)
                - fric * (jnp.float32(1.0) + elas) * jnp.abs(nvz) / vtn)
            cvz = jnp.where(hit, -elas * nvz, nvz)
            cvx = jnp.where(hit, a * nvx, nvx)
            cvy = jnp.where(hit, a * nvy, nvy)
            cxz = jnp.maximum(nxz, jnp.float32(COLLISION_DIST))
            plsc.store_scatter(xvbuf, [r, _c(0)], nxx)
            plsc.store_scatter(xvbuf, [r, _c(1)], nxy)
            plsc.store_scatter(xvbuf, [r, _c(2)], cxz)
            plsc.store_scatter(xvbuf, [r, _c(3)], cvx)
            plsc.store_scatter(xvbuf, [r, _c(4)], cvy)
            plsc.store_scatter(xvbuf, [r, _c(5)], cvz)
            return carry2

        lax.fori_loop(0, RPT // _LANES, vrow, 0)
        pltpu.sync_copy(xvbuf, xv_out.at[pl.ds(wid * RPT, RPT)])

    return pl.kernel(body, out_type=tuple(out_type), mesh=mesh,
                     scratch_types=scratch,
                     compiler_params=pltpu.CompilerParams(
                         needs_layout_passes=False,
                         use_tc_tiling_on_sc=False))


def kernel(init_vertices, init_springs, init_rest_lengths, init_masses,
           spring_Y_log, collide_elas, collide_fric):
    f32 = jnp.float32
    i32 = jnp.int32
    xpad = jnp.zeros((VP - N_VERT, 3), f32)
    x0 = jnp.concatenate([init_vertices.astype(f32), xpad], axis=0)
    mpad = jnp.ones((VP - N_VERT,), f32)
    m0 = jnp.concatenate([init_masses.astype(f32), mpad], axis=0)
    xv0 = jnp.concatenate(
        [x0, jnp.zeros((VP, 3), f32), m0[:, None],
         jnp.zeros((VP, XVW - 7), f32)], axis=1)

    i1 = init_springs[:, 0].astype(i32)
    i2 = init_springs[:, 1].astype(i32)
    dstA = jnp.concatenate([i1, i2])
    srcA = jnp.concatenate([i2, i1])
    e = jnp.arange(2 * N_SPR, dtype=i32)
    order = jnp.argsort(dstA)
    dstS = dstA[order]
    srcS = srcA[order]
    sidS = jnp.where(order < N_SPR, order, order - N_SPR)
    ylogS = spring_Y_log.astype(f32)[sidS]
    restS = init_rest_lengths.astype(f32)[sidS]
    osS = jnp.where(order < N_SPR, order, N_SPR + (e % SF_PAD))

    bounds = jnp.searchsorted(
        dstS, jnp.arange(NW + 1, dtype=i32) * RPT).astype(i32)
    pos = bounds[:NW, None] + jnp.arange(NPT, dtype=i32)[None, :]
    valid = pos < bounds[1:, None]
    posc = jnp.minimum(pos, 2 * N_SPR - 1)
    dfl = jnp.arange(NPT, dtype=i32)[None, :] % RPT
    wbase = (jnp.arange(NW, dtype=i32) * RPT)[:, None]
    srcT = jnp.where(valid, srcS[posc], wbase + dfl)
    dstlT = jnp.where(valid, dstS[posc] - wbase, dfl)
    ylogT = jnp.where(valid, ylogS[posc], jnp.float32(0.0))
    restT = jnp.where(valid, restS[posc], jnp.float32(1.0))
    osT = jnp.where(valid, osS[posc], N_SPR + (pos % SF_PAD))
    ebg = jnp.stack(
        [dstlT.astype(f32), ylogT, restT, jnp.zeros_like(ylogT)], axis=-1)
    srcg = srcT.reshape(NW, NPT // 128, 128)
    osg = osT.reshape(NW, NPT // 128, 128)

    z4 = jnp.zeros((VP * 4,), f32)
    prm = jnp.concatenate([
        jnp.full((16,), collide_elas, f32),
        jnp.full((16,), collide_fric, f32)])

    step = _make_step(False)
    step_sf = _make_step(True)
    xv = xv0
    for _ in range(NUM_SUBSTEPS - 1):
        (xv,) = step(xv, srcg, ebg, osg, z4, prm)
    xv, sf = step_sf(xv, srcg, ebg, osg, z4, prm)

    x_final = xv[:N_VERT, :3]
    spring_forces = sf[:N_SPR, :3]
    return (x_final, init_springs, init_rest_lengths, spring_forces)


# speedup vs baseline: 3.1514x; 1.0153x over previous
import math

import jax
import jax.numpy as jnp
from jax import lax
from jax.experimental import pallas as pl
from jax.experimental.pallas import tpu as pltpu
from jax.experimental.pallas import tpu_sc as plsc

N_VERT = 50000
N_SPR = 800000
DT = 0.001
NUM_SUBSTEPS = 10
DASHPOT_DAMPING = 0.1
DRAG_DAMPING = 0.5
COLLISION_DIST = 0.05

NW = 32
VP = 50176
RPT = VP // NW
EB = 1536
NBLK = 36
NPT = EB * NBLK
KSUB = EB // 128
XVW = 16
SF_PAD = 8192
NSF = N_SPR + SF_PAD

_DAMP = math.exp(-DT * DRAG_DAMPING)
_LANES = 16


def _rsqrt(x):
    xs = jnp.maximum(x, jnp.float32(1e-30))
    i = lax.bitcast_convert_type(xs, jnp.int32)
    i = jnp.int32(0x5F3759DF) - lax.shift_right_logical(i, 1)
    y = lax.bitcast_convert_type(i, jnp.float32)
    for _ in range(3):
        y = y * (jnp.float32(1.5) - jnp.float32(0.5) * xs * y * y)
    return y


def _c(v):
    return jnp.full((_LANES,), v, jnp.int32)


def _make_step(emit_sf: bool):
    mesh = plsc.VectorSubcoreMesh(core_axis_name="c", subcore_axis_name="s",
                                  num_cores=2, num_subcores=16)
    out_type = [jax.ShapeDtypeStruct((VP, XVW), jnp.float32)]
    if emit_sf:
        out_type.append(jax.ShapeDtypeStruct((NSF, 8), jnp.float32))
    scratch = [
        pltpu.VMEM((KSUB, 128), jnp.int32),
        pltpu.VMEM((KSUB, 128), jnp.int32),
        pltpu.VMEM((KSUB, 128), jnp.int32),
        pltpu.VMEM((EB, 4), jnp.float32),
        pltpu.VMEM((EB, 4), jnp.float32),
        pltpu.VMEM((EB, XVW), jnp.float32),
        pltpu.VMEM((EB, XVW), jnp.float32),
        pltpu.VMEM((EB, 8), jnp.float32),
        pltpu.VMEM((RPT * 4,), jnp.float32),
        pltpu.VMEM((RPT, XVW), jnp.float32),
        pltpu.VMEM((32,), jnp.float32),
        pltpu.SemaphoreType.DMA,
        pltpu.SemaphoreType.DMA,
    ]

    def body(xv_in, srcg, ebg, osg, z4, prm, *rest):
        if emit_sf:
            xv_out, sf_out = rest[0], rest[1]
            scr = rest[2:]
        else:
            xv_out = rest[0]
            scr = rest[1:]
        (i1bufA, i1bufB, osbuf, ebufA, ebufB, rows1A, rows1B, sfbuf,
         accf, xvbuf, pbuf, semA, semB) = scr
        cid = lax.axis_index("c")
        sid = lax.axis_index("s")
        wid = cid * 16 + sid
        iota = lax.iota(jnp.int32, _LANES)

        pltpu.sync_copy(xv_in.at[pl.ds(wid * RPT, RPT)], xvbuf)
        pltpu.sync_copy(z4.at[pl.ds(wid * RPT * 4, RPT * 4)], accf)
        pltpu.sync_copy(prm, pbuf)

        def fire(b, i1buf, ebuf, rows1, sem):
            pltpu.sync_copy(srcg.at[wid, pl.ds(b * KSUB, KSUB)], i1buf)
            pltpu.sync_copy(ebg.at[wid, pl.ds(b * EB, EB)], ebuf)
            for kb in range(KSUB):
                pltpu.async_copy(xv_in.at[i1buf.at[kb]],
                                 rows1.at[pl.ds(kb * 128, 128)], sem)

        def drain(i1buf, rows1, sem):
            for kb in range(KSUB):
                pltpu.make_async_copy(xv_in.at[i1buf.at[kb]],
                                      rows1.at[pl.ds(kb * 128, 128)],
                                      sem).wait()

        def compute(b, i1buf, ebuf, rows1):
            if emit_sf:
                pltpu.sync_copy(osg.at[wid, pl.ds(b * KSUB, KSUB)], osbuf)

            def vec(j, carry2):
                r = j * _LANES + iota
                dstl = plsc.load_gather(ebuf, [r, _c(0)]).astype(jnp.int32)
                ylv = plsc.load_gather(ebuf, [r, _c(1)])
                rv = plsc.load_gather(ebuf, [r, _c(2)])
                x1x = plsc.load_gather(xvbuf, [dstl, _c(0)])
                x1y = plsc.load_gather(xvbuf, [dstl, _c(1)])
                x1z = plsc.load_gather(xvbuf, [dstl, _c(2)])
                v1x = plsc.load_gather(xvbuf, [dstl, _c(3)])
                v1y = plsc.load_gather(xvbuf, [dstl, _c(4)])
                v1z = plsc.load_gather(xvbuf, [dstl, _c(5)])
                x2x = plsc.load_gather(rows1, [r, _c(0)])
                x2y = plsc.load_gather(rows1, [r, _c(1)])
                x2z = plsc.load_gather(rows1, [r, _c(2)])
                v2x = plsc.load_gather(rows1, [r, _c(3)])
                v2y = plsc.load_gather(rows1, [r, _c(4)])
                v2z = plsc.load_gather(rows1, [r, _c(5)])
                Y = jnp.exp(ylv)
                ir = jnp.float32(1.0) / rv
                dx = x2x - x1x
                dy = x2y - x1y
                dz = x2z - x1z
                L2 = dx * dx + dy * dy + dz * dz
                ri = _rsqrt(L2)
                L = L2 * ri
                ddx = dx * ri
                ddy = dy * ri
                ddz = dz * ri
                scoef = Y * (L * ir - jnp.float32(1.0))
                vrel = ((v2x - v1x) * ddx + (v2y - v1y) * ddy
                        + (v2z - v1z) * ddz)
                fc = scoef + jnp.float32(DASHPOT_DAMPING) * vrel
                a4 = dstl * 4
                plsc.addupdate_scatter(accf, [a4], fc * ddx)
                plsc.addupdate_scatter(accf, [a4 + 1], fc * ddy)
                plsc.addupdate_scatter(accf, [a4 + 2], fc * ddz)
                if emit_sf:
                    plsc.store_scatter(sfbuf, [r, _c(0)], scoef * ddx)
                    plsc.store_scatter(sfbuf, [r, _c(1)], scoef * ddy)
                    plsc.store_scatter(sfbuf, [r, _c(2)], scoef * ddz)
                return carry2

            lax.fori_loop(0, EB // _LANES, vec, 0)

            if emit_sf:
                sps = [pltpu.async_copy(
                           sfbuf.at[pl.ds(kb * 128, 128)],
                           sf_out.at[osbuf.at[kb]], semA)
                       for kb in range(KSUB)]
                for cp in sps:
                    cp.wait()

        fire(0, i1bufA, ebufA, rows1A, semA)

        def pair(k, carry):
            b0 = 2 * k
            fire(b0 + 1, i1bufB, ebufB, rows1B, semB)
            drain(i1bufA, rows1A, semA)
            compute(b0, i1bufA, ebufA, rows1A)

            @pl.when(k < NBLK // 2 - 1)
            def _():
                fire(b0 + 2, i1bufA, ebufA, rows1A, semA)
            drain(i1bufB, rows1B, semB)
            compute(b0 + 1, i1bufB, ebufB, rows1B)
            return carry

        lax.fori_loop(0, NBLK // 2, pair, 0)

        elas = jnp.clip(pbuf[pl.ds(0, _LANES)], 0.0, 1.0)
        fric = jnp.clip(pbuf[pl.ds(16, _LANES)], 0.0, 1.0)

        def vrow(j, carry2):
            r = j * _LANES + iota
            r4 = r * 4
            fx = plsc.load_gather(accf, [r4])
            fy = plsc.load_gather(accf, [r4 + 1])
            fz = plsc.load_gather(accf, [r4 + 2])
            xx = plsc.load_gather(xvbuf, [r, _c(0)])
            xy = plsc.load_gather(xvbuf, [r, _c(1)])
            xz = plsc.load_gather(xvbuf, [r, _c(2)])
            vx = plsc.load_gather(xvbuf, [r, _c(3)])
            vy = plsc.load_gather(xvbuf, [r, _c(4)])
            vz = plsc.load_gather(xvbuf, [r, _c(5)])
            m = plsc.load_gather(xvbuf, [r, _c(6)])
            minv = jnp.float32(1.0) / m
            nvx = (vx + jnp.float32(DT) * fx * minv) * jnp.float32(_DAMP)
            nvy = (vy + jnp.float32(DT) * fy * minv) * jnp.float32(_DAMP)
            nvz = ((vz + jnp.float32(DT) * (fz * minv - jnp.float32(9.8)))
                   * jnp.float32(_DAMP))
            nxx = xx + jnp.float32(DT) * nvx
            nxy = xy + jnp.float32(DT) * nvy
            nxz = xz + jnp.float32(DT) * nvz
            hit = (nxz < jnp.float32(COLLISION_DIST)) & (nvz < jnp.float32(0.0))
            vt2 = nvx * nvx + nvy * nvy
            vtn = vt2 * _rsqrt(vt2) + jnp.float32(1e-8)
            a = jnp.maximum(
                jnp.float32(0.0),
                jnp.float32(1.0)
                - fric * (jnp.float32(1.0) + elas) * jnp.abs(nvz) / vtn)
            cvz = jnp.where(hit, -elas * nvz, nvz)
            cvx = jnp.where(hit, a * nvx, nvx)
            cvy = jnp.where(hit, a * nvy, nvy)
            cxz = jnp.maximum(nxz, jnp.float32(COLLISION_DIST))
            plsc.store_scatter(xvbuf, [r, _c(0)], nxx)
            plsc.store_scatter(xvbuf, [r, _c(1)], nxy)
            plsc.store_scatter(xvbuf, [r, _c(2)], cxz)
            plsc.store_scatter(xvbuf, [r, _c(3)], cvx)
            plsc.store_scatter(xvbuf, [r, _c(4)], cvy)
            plsc.store_scatter(xvbuf, [r, _c(5)], cvz)
            return carry2

        lax.fori_loop(0, RPT // _LANES, vrow, 0)
        pltpu.sync_copy(xvbuf, xv_out.at[pl.ds(wid * RPT, RPT)])

    return pl.kernel(body, out_type=tuple(out_type), mesh=mesh,
                     scratch_types=scratch,
                     compiler_params=pltpu.CompilerParams(
                         needs_layout_passes=False,
                         use_tc_tiling_on_sc=False))


def kernel(init_vertices, init_springs, init_rest_lengths, init_masses,
           spring_Y_log, collide_elas, collide_fric):
    f32 = jnp.float32
    i32 = jnp.int32
    xpad = jnp.zeros((VP - N_VERT, 3), f32)
    x0 = jnp.concatenate([init_vertices.astype(f32), xpad], axis=0)
    mpad = jnp.ones((VP - N_VERT,), f32)
    m0 = jnp.concatenate([init_masses.astype(f32), mpad], axis=0)
    xv0 = jnp.concatenate(
        [x0, jnp.zeros((VP, 3), f32), m0[:, None],
         jnp.zeros((VP, XVW - 7), f32)], axis=1)

    i1 = init_springs[:, 0].astype(i32)
    i2 = init_springs[:, 1].astype(i32)
    dstA = jnp.concatenate([i1, i2])
    srcA = jnp.concatenate([i2, i1])
    e = jnp.arange(2 * N_SPR, dtype=i32)
    order = jnp.argsort(dstA)
    dstS = dstA[order]
    srcS = srcA[order]
    sidS = jnp.where(order < N_SPR, order, order - N_SPR)
    ylogS = spring_Y_log.astype(f32)[sidS]
    restS = init_rest_lengths.astype(f32)[sidS]
    osS = jnp.where(order < N_SPR, order, N_SPR + (e % SF_PAD))

    bounds = jnp.searchsorted(
        dstS, jnp.arange(NW + 1, dtype=i32) * RPT).astype(i32)
    pos = bounds[:NW, None] + jnp.arange(NPT, dtype=i32)[None, :]
    valid = pos < bounds[1:, None]
    posc = jnp.minimum(pos, 2 * N_SPR - 1)
    dfl = jnp.arange(NPT, dtype=i32)[None, :] % RPT
    wbase = (jnp.arange(NW, dtype=i32) * RPT)[:, None]
    srcT = jnp.where(valid, srcS[posc], wbase + dfl)
    dstlT = jnp.where(valid, dstS[posc] - wbase, dfl)
    ylogT = jnp.where(valid, ylogS[posc], jnp.float32(0.0))
    restT = jnp.where(valid, restS[posc], jnp.float32(1.0))
    osT = jnp.where(valid, osS[posc], N_SPR + (pos % SF_PAD))
    ebg = jnp.stack(
        [dstlT.astype(f32), ylogT, restT, jnp.zeros_like(ylogT)], axis=-1)
    srcg = srcT.reshape(NW, NPT // 128, 128)
    osg = osT.reshape(NW, NPT // 128, 128)

    z4 = jnp.zeros((VP * 4,), f32)
    prm = jnp.concatenate([
        jnp.full((16,), collide_elas, f32),
        jnp.full((16,), collide_fric, f32)])

    step = _make_step(False)
    step_sf = _make_step(True)
    xv = xv0
    for _ in range(NUM_SUBSTEPS - 1):
        (xv,) = step(xv, srcg, ebg, osg, z4, prm)
    xv, sf = step_sf(xv, srcg, ebg, osg, z4, prm)

    x_final = xv[:N_VERT, :3]
    spring_forces = sf[:N_SPR, :3]
    return (x_final, init_springs, init_rest_lengths, spring_forces)
